# baseline (speedup 1.0000x reference)
import functools
import jax
import jax.numpy as jnp
from jax import lax
from jax.experimental import pallas as pl
from jax.experimental.pallas import tpu as pltpu
from jax.experimental.pallas import tpu_sc as plsc


def kernel(x, row_embed):
    B = x.shape[0]
    W, D = row_embed.shape
    NC, NS = 2, 16
    NW = NC * NS
    bpw = B // NW
    K = 4
    ND = bpw // K

    mesh = plsc.VectorSubcoreMesh(
        core_axis_name="c", subcore_axis_name="s", num_cores=NC, num_subcores=NS
    )

    @functools.partial(
        pl.kernel,
        mesh=mesh,
        out_type=jax.ShapeDtypeStruct((B, W, D), jnp.float32),
        scratch_types=[
            pltpu.VMEM((K, W, D), jnp.float32),
            pltpu.SemaphoreType.DMA,
        ],
    )
    def sc_broadcast(row_hbm, out_hbm, rep_v, sem):
        wid = lax.axis_index("s") * NC + lax.axis_index("c")
        base = wid * bpw
        for j in range(K):
            pltpu.sync_copy(row_hbm, rep_v.at[j])
        cps = [
            pltpu.async_copy(rep_v, out_hbm.at[pl.ds(base + i * K, K)], sem)
            for i in range(ND)
        ]
        for cp in cps:
            cp.wait()

    return sc_broadcast(row_embed)


# speedup vs baseline: 1.0017x; 1.0017x over previous
import functools
import jax
import jax.numpy as jnp
from jax import lax
from jax.experimental import pallas as pl
from jax.experimental.pallas import tpu as pltpu
from jax.experimental.pallas import tpu_sc as plsc


def kernel(x, row_embed):
    B = x.shape[0]
    W, D = row_embed.shape
    NC, NS = 2, 16
    NW = NC * NS
    bpw = B // NW
    K = 4
    ND = bpw // K

    mesh = plsc.VectorSubcoreMesh(
        core_axis_name="c", subcore_axis_name="s", num_cores=NC, num_subcores=NS
    )

    @functools.partial(
        pl.kernel,
        mesh=mesh,
        out_type=jax.ShapeDtypeStruct((B, W, D), jnp.float32),
        scratch_types=[
            pltpu.VMEM((K, W, D), jnp.float32),
            pltpu.SemaphoreType.DMA,
        ],
        compiler_params=pltpu.CompilerParams(use_tc_tiling_on_sc=True),
    )
    def sc_broadcast(row_hbm, out_hbm, rep_v, sem):
        wid = lax.axis_index("s") * NC + lax.axis_index("c")
        base = wid * bpw
        for j in range(K):
            pltpu.sync_copy(row_hbm, rep_v.at[j])
        cps = [
            pltpu.async_copy(rep_v, out_hbm.at[pl.ds(base + i * K, K)], sem)
            for i in range(ND)
        ]
        for cp in cps:
            cp.wait()

    return sc_broadcast(row_embed)


# speedup vs baseline: 2.4782x; 2.4740x over previous
import functools
import jax
import jax.numpy as jnp
from jax import lax
from jax.experimental import pallas as pl
from jax.experimental.pallas import tpu as pltpu
from jax.experimental.pallas import tpu_sc as plsc


def kernel(x, row_embed):
    B = x.shape[0]
    W, D = row_embed.shape
    NC, NS = 2, 16
    NW = NC * NS
    bpw = B // NW
    RB = 8
    HW = W // 2
    NCH = bpw // RB

    rep_all = jnp.broadcast_to(
        row_embed.reshape(2, HW, 1, D), (2, HW, RB, D)
    )

    mesh = plsc.VectorSubcoreMesh(
        core_axis_name="c", subcore_axis_name="s", num_cores=NC, num_subcores=NS
    )

    @functools.partial(
        pl.kernel,
        mesh=mesh,
        out_type=jax.ShapeDtypeStruct((W, B, D), jnp.float32),
        scratch_types=[
            pltpu.VMEM((HW, RB, D), jnp.float32),
            pltpu.SemaphoreType.DMA,
        ],
    )
    def sc_broadcast(rep_hbm, out_hbm, rep_v, sem):
        wid = lax.axis_index("s") * NC + lax.axis_index("c")
        base = wid * bpw
        for h in range(2):
            pltpu.sync_copy(rep_hbm.at[h], rep_v)
            cps = [
                pltpu.async_copy(
                    rep_v,
                    out_hbm.at[pl.ds(h * HW, HW), pl.ds(base + i * RB, RB)],
                    sem,
                )
                for i in range(NCH)
            ]
            for cp in cps:
                cp.wait()

    return jnp.transpose(sc_broadcast(rep_all), (1, 0, 2))


# speedup vs baseline: 2.5660x; 1.0354x over previous
import functools
import jax
import jax.numpy as jnp
from jax import lax
from jax.experimental import pallas as pl
from jax.experimental.pallas import tpu as pltpu
from jax.experimental.pallas import tpu_sc as plsc


def kernel(x, row_embed):
    B = x.shape[0]
    W, D = row_embed.shape
    NC, NS = 2, 16
    NW = NC * NS
    bpw = B // NW
    RB = 8
    HW = W // 2
    NCH = bpw // RB

    rep_all = jnp.broadcast_to(
        row_embed.reshape(2, HW, 1, D), (2, HW, RB, D)
    )

    mesh = plsc.VectorSubcoreMesh(
        core_axis_name="c", subcore_axis_name="s", num_cores=NC, num_subcores=NS
    )

    @functools.partial(
        pl.kernel,
        mesh=mesh,
        out_type=jax.ShapeDtypeStruct((W, B, D), jnp.float32),
        scratch_types=[
            pltpu.VMEM((HW, RB, D), jnp.float32),
            pltpu.VMEM_SHARED((HW, RB, D), jnp.float32),
            pltpu.SemaphoreType.DMA,
            pltpu.SemaphoreType.DMA,
        ],
    )
    def sc_broadcast(rep_hbm, out_hbm, rep_v, shr_v, sem_t, sem_s):
        c = lax.axis_index("c")
        s = lax.axis_index("s")
        wid = s * NC + c
        base = wid * bpw
        F = 6
        for h in range(2):
            @pl.when(s == 0)
            def _stage_shared():
                pltpu.sync_copy(rep_hbm.at[h], shr_v)

            pltpu.sync_copy(rep_hbm.at[h], rep_v)
            plsc.subcore_barrier()
            cps = []
            for i in range(NCH):
                src, sem = (shr_v, sem_s) if i < F else (rep_v, sem_t)
                cps.append(
                    pltpu.async_copy(
                        src,
                        out_hbm.at[pl.ds(h * HW, HW), pl.ds(base + i * RB, RB)],
                        sem,
                    )
                )
            for cp in cps:
                cp.wait()
            plsc.subcore_barrier()

    return jnp.transpose(sc_broadcast(rep_all), (1, 0, 2))
